# Initial kernel scaffold; baseline (speedup 1.0000x reference)
import functools

import jax
import jax.numpy as jnp
from jax import lax
from jax.experimental import pallas as pl
from jax.experimental.pallas import tpu as pltpu
from jax.experimental.pallas import tpu_sc as plsc

N = 10000
E = 160000
G = 16
R = 1000
RT = N // 16
F32 = jnp.float32


def _make_prop(F, n_chunks, edge_split, nb):
    mesh = plsc.VectorSubcoreMesh(core_axis_name="c", subcore_axis_name="s")
    passes = 1 if edge_split else n_chunks // 2
    out_shape = (2, N, F) if edge_split else (n_chunks, N, F)
    scratch = [
        pltpu.VMEM((nb, 128), jnp.int32),
        pltpu.VMEM((nb, 128), jnp.int32),
        pltpu.VMEM((nb, 128), jnp.int32),
        pltpu.VMEM((128, F), F32),
        pltpu.VMEM_SHARED((N + 16, F), F32),
    ]

    @functools.partial(
        pl.kernel, mesh=mesh,
        out_type=jax.ShapeDtypeStruct(out_shape, F32),
        scratch_types=scratch)
    def prop(y, srcb, dstb, out, src_v, src2_v, dst_v, rows_v, acc):
        cid = lax.axis_index("c")
        sid = lax.axis_index("s")
        if edge_split:
            pltpu.sync_copy(srcb.at[cid, sid], src_v)
            pltpu.sync_copy(dstb.at[cid, sid], dst_v)
        else:
            pltpu.sync_copy(srcb.at[sid], src_v)
            pltpu.sync_copy(dstb.at[sid], dst_v)
        zeros16 = jnp.zeros((16,), F32)
        for p in range(passes):
            ci = cid * passes + p

            def zrow(r, _):
                def zcol(k, _2):
                    rows_v[r, pl.ds(k * 16, 16)] = zeros16
                    return 0
                return lax.fori_loop(0, F // 16, zcol, 0)
            lax.fori_loop(0, 128, zrow, 0)
            for t in range(5):
                pltpu.sync_copy(rows_v.at[pl.ds(0, 125)],
                                acc.at[pl.ds(sid * RT + t * 125, 125)])
            if not edge_split:
                def arow(r, _):
                    def acol(k, _2):
                        sl = pl.ds(k * 16, 16)
                        src2_v[r, sl] = src_v[r, sl] + ci * N
                        return 0
                    return lax.fori_loop(0, 8, acol, 0)
                lax.fori_loop(0, nb, arow, 0)
            sidx = src_v if edge_split else src2_v
            plsc.subcore_barrier()

            def eblk(j, _):
                pltpu.sync_copy(y.at[sidx.at[j]], rows_v)
                pltpu.sync_copy(rows_v, acc.at[dst_v.at[j]], add=True)
                return 0
            lax.fori_loop(0, nb, eblk, 0)
            plsc.subcore_barrier()
            for t in range(5):
                pltpu.sync_copy(acc.at[pl.ds(sid * RT + t * 125, 125)],
                                rows_v.at[pl.ds(0, 125)])
                oi = cid if edge_split else ci
                pltpu.sync_copy(rows_v.at[pl.ds(0, 125)],
                                out.at[oi, pl.ds(sid * RT + t * 125, 125)])
            plsc.subcore_barrier()

    return prop


_prop_es16 = _make_prop(16, 1, True, 40)
_prop_ch128 = _make_prop(128, 4, False, 79)


def _pad_edges_es(v, pad_val):
    per = 2 * 16 * 40 * 128
    vp = jnp.concatenate([v, jnp.full((per - E,), pad_val, jnp.int32)])
    return vp.reshape(2, 16, 40, 128)


def _pad_edges_ch(v, pad_val):
    per = 16 * 79 * 128
    vp = jnp.concatenate([v, jnp.full((per - E,), pad_val, jnp.int32)])
    return vp.reshape(16, 79, 128)


def _kpre_body(dp, x16, dis, y0):
    p = dp[0] + dp[1]
    deg = 1.0 + p[:, 0:1]
    d = lax.rsqrt(deg)
    dis[...] = d
    y0[...] = x16[...] * d


def _kpre(dp, x16):
    return pl.pallas_call(
        _kpre_body,
        grid=(N // R,),
        in_specs=[
            pl.BlockSpec((2, R, 16), lambda i: (0, i, 0)),
            pl.BlockSpec((R, 16), lambda i: (i, 0)),
        ],
        out_specs=[
            pl.BlockSpec((R, 1), lambda i: (i, 0)),
            pl.BlockSpec((R, 16), lambda i: (i, 0)),
        ],
        out_shape=[
            jax.ShapeDtypeStruct((N, 1), F32),
            jax.ShapeDtypeStruct((N, 16), F32),
        ],
    )(dp, x16)


def _k1_body(zp, y0, dis, w, b, out):
    d = dis[...]
    u = d * (zp[0] + zp[1] + y0[...])
    h = jnp.maximum(jnp.dot(u, w[...], preferred_element_type=F32) + b[...], 0.0)
    out[0] = d * h


def _k1(zp, y0, dis, w1p, b1):
    return pl.pallas_call(
        _k1_body,
        grid=(N // R, 4),
        in_specs=[
            pl.BlockSpec((2, R, 16), lambda i, c: (0, i, 0)),
            pl.BlockSpec((R, 16), lambda i, c: (i, 0)),
            pl.BlockSpec((R, 1), lambda i, c: (i, 0)),
            pl.BlockSpec((16, 128), lambda i, c: (0, c)),
            pl.BlockSpec((128,), lambda i, c: (c,)),
        ],
        out_specs=pl.BlockSpec((1, R, 128), lambda i, c: (c, i, 0)),
        out_shape=jax.ShapeDtypeStruct((4, N, 128), F32),
    )(zp, y0, dis, w1p, b1)


def _k2a_body(z1, y1, dis, w2r, b2, out):
    d = dis[...]
    acc = jnp.zeros((R, 1024), F32)
    for c in range(4):
        u = d * (z1[c] + y1[c])
        acc = acc + jnp.dot(u, w2r[c], preferred_element_type=F32)
    out[...] = jnp.maximum(acc + b2[...], 0.0)


def _k2a(z1, y1, dis, w2r, b2):
    return pl.pallas_call(
        _k2a_body,
        grid=(N // R,),
        in_specs=[
            pl.BlockSpec((4, R, 128), lambda i: (0, i, 0)),
            pl.BlockSpec((4, R, 128), lambda i: (0, i, 0)),
            pl.BlockSpec((R, 1), lambda i: (i, 0)),
            pl.BlockSpec((4, 128, 1024), lambda i: (0, 0, 0)),
            pl.BlockSpec((1024,), lambda i: (0,)),
        ],
        out_specs=pl.BlockSpec((R, 1024), lambda i: (i, 0)),
        out_shape=jax.ShapeDtypeStruct((N, 1024), F32),
    )(z1, y1, dis, w2r, b2)


def _k2b_body(h2, w3, dis, out):
    m = jnp.dot(h2[...], w3[...], preferred_element_type=F32)
    out[0] = dis[...] * m


def _k2b(h2, w3, dis):
    return pl.pallas_call(
        _k2b_body,
        grid=(N // R, 4),
        in_specs=[
            pl.BlockSpec((R, 1024), lambda i, c: (i, 0)),
            pl.BlockSpec((1024, 128), lambda i, c: (0, c)),
            pl.BlockSpec((R, 1), lambda i, c: (i, 0)),
        ],
        out_specs=pl.BlockSpec((1, R, 128), lambda i, c: (c, i, 0)),
        out_shape=jax.ShapeDtypeStruct((4, N, 128), F32),
    )(h2, w3, dis)


def _k3_body(z2, y2, dis, batch, b3, fc1w, fc1b, fc2p, fc2b, out,
             sums, cnt):
    i = pl.program_id(0)

    @pl.when(i == 0)
    def _init():
        sums[...] = jnp.zeros((G, 512), F32)
        cnt[...] = jnp.zeros((G, 8), F32)

    d = dis[...]
    oh = (batch[...] == lax.broadcasted_iota(jnp.int32, (R, G), 1)).astype(F32)
    dn = (((0,), (0,)), ((), ()))
    for c in range(4):
        u = d * (z2[c] + y2[c]) + b3[pl.ds(c * 128, 128)]
        h3c = jnp.maximum(u, 0.0)
        sums[:, pl.ds(c * 128, 128)] += lax.dot_general(
            oh, h3c, dn, preferred_element_type=F32)
    cnt[...] += lax.dot_general(oh, jnp.ones((R, 8), F32), dn,
                                preferred_element_type=F32)

    @pl.when(i == N // R - 1)
    def _fin():
        denom = jnp.maximum(cnt[:, 0:1], 1.0)
        g = sums[...] / denom
        a = jnp.maximum(
            jnp.dot(g, fc1w[...], preferred_element_type=F32) + fc1b[...], 0.0)
        out[...] = jnp.dot(a, fc2p[...], preferred_element_type=F32) + fc2b[...]


def _k3(z2, y2, dis, batchc, b3, fc1w, fc1b, fc2p, fc2b):
    return pl.pallas_call(
        _k3_body,
        grid=(N // R,),
        in_specs=[
            pl.BlockSpec((4, R, 128), lambda i: (0, i, 0)),
            pl.BlockSpec((4, R, 128), lambda i: (0, i, 0)),
            pl.BlockSpec((R, 1), lambda i: (i, 0)),
            pl.BlockSpec((R, 1), lambda i: (i, 0)),
            pl.BlockSpec((512,), lambda i: (0,)),
            pl.BlockSpec((512, 128), lambda i: (0, 0)),
            pl.BlockSpec((128,), lambda i: (0,)),
            pl.BlockSpec((128, 128), lambda i: (0, 0)),
            pl.BlockSpec((1,), lambda i: (0,)),
        ],
        out_specs=pl.BlockSpec((G, 128), lambda i: (0, 0)),
        out_shape=jax.ShapeDtypeStruct((G, 128), F32),
        scratch_shapes=[
            pltpu.VMEM((G, 512), F32),
            pltpu.VMEM((G, 8), F32),
        ],
    )(z2, y2, dis, batchc, b3, fc1w, fc1b, fc2p, fc2b)


def kernel(x, edge_index, batch, W1, b1, W2, b2, W3, b3,
           fc1_w, fc1_b, fc2_w, fc2_b):
    src = edge_index[0].astype(jnp.int32)
    dst = edge_index[1].astype(jnp.int32)
    src_es = _pad_edges_es(src, 0)
    dst_es = _pad_edges_es(dst, N)
    src_ch = _pad_edges_ch(src, 0)
    dst_ch = _pad_edges_ch(dst, N)

    ones16 = jnp.ones((N, 16), F32)
    x16 = jnp.pad(x, ((0, 0), (0, 13)))
    w1p = jnp.pad(W1, ((0, 13), (0, 0)))
    w2r = W2.reshape(4, 128, 1024)
    fc2p = jnp.pad(fc2_w, ((0, 0), (0, 127)))
    batchc = batch.astype(jnp.int32)[:, None]

    dp = _prop_es16(ones16, src_es, dst_es)
    dis, y0 = _kpre(dp, x16)
    zp = _prop_es16(y0, src_es, dst_es)
    y1 = _k1(zp, y0, dis, w1p, b1)
    z1 = _prop_ch128(y1.reshape(4 * N, 128), src_ch, dst_ch)
    h2 = _k2a(z1, y1, dis, w2r, b2)
    y2 = _k2b(h2, W3, dis)
    z2 = _prop_ch128(y2.reshape(4 * N, 128), src_ch, dst_ch)
    o = _k3(z2, y2, dis, batchc, b3, fc1_w, fc1_b, fc2p, fc2b)
    return o[:, 0:1]


# speedup vs baseline: 8.0913x; 8.0913x over previous
import functools

import jax
import jax.numpy as jnp
from jax import lax
from jax.experimental import pallas as pl
from jax.experimental.pallas import tpu as pltpu
from jax.experimental.pallas import tpu_sc as plsc

N = 10000
E = 160000
G = 16
R = 1000
NP = 10240
RT = NP // 16
F32 = jnp.float32


def _make_prop(F, n_chunks, edge_split, nb, ones_src=False):
    mesh = plsc.VectorSubcoreMesh(core_axis_name="c", subcore_axis_name="s")
    passes = 1 if edge_split else n_chunks // 2
    out_shape = (2, NP, F) if edge_split else (n_chunks, NP, F)
    scratch = [
        pltpu.VMEM((nb, 128), jnp.int32),
        pltpu.VMEM((nb, 128), jnp.int32),
        pltpu.VMEM((nb, 128), jnp.int32),
        pltpu.VMEM((128, F), F32),
        pltpu.VMEM_SHARED((NP, F), F32),
    ]

    @functools.partial(
        pl.kernel, mesh=mesh,
        out_type=jax.ShapeDtypeStruct(out_shape, F32),
        scratch_types=scratch)
    def prop(y, srcb, dstb, out, src_v, src2_v, dst_v, rows_v, acc):
        cid = lax.axis_index("c")
        sid = lax.axis_index("s")
        if edge_split:
            if not ones_src:
                pltpu.sync_copy(srcb.at[cid, sid], src_v)
            pltpu.sync_copy(dstb.at[cid, sid], dst_v)
        else:
            pltpu.sync_copy(srcb.at[sid], src_v)
            pltpu.sync_copy(dstb.at[sid], dst_v)
        zeros16 = jnp.zeros((16,), F32)
        ones16v = jnp.ones((16,), F32)
        for p in range(passes):
            ci = cid * passes + p

            def zrow(r, _):
                def zcol(k, _2):
                    rows_v[r, pl.ds(k * 16, 16)] = zeros16
                    return 0
                return lax.fori_loop(0, F // 16, zcol, 0)
            lax.fori_loop(0, 128, zrow, 0)
            for t in range(5):
                pltpu.sync_copy(rows_v,
                                acc.at[pl.ds(sid * RT + t * 128, 128)])
            if ones_src:
                def orow(r, _):
                    def ocol(k, _2):
                        rows_v[r, pl.ds(k * 16, 16)] = ones16v
                        return 0
                    return lax.fori_loop(0, F // 16, ocol, 0)
                lax.fori_loop(0, 128, orow, 0)
            if not edge_split:
                def arow(r, _):
                    def acol(k, _2):
                        sl = pl.ds(k * 16, 16)
                        src2_v[r, sl] = src_v[r, sl] + ci * N
                        return 0
                    return lax.fori_loop(0, 8, acol, 0)
                lax.fori_loop(0, nb, arow, 0)
            sidx = src_v if edge_split else src2_v
            plsc.subcore_barrier()

            def eblk(j, _):
                if not ones_src:
                    pltpu.sync_copy(y.at[sidx.at[j]], rows_v)
                pltpu.sync_copy(rows_v, acc.at[dst_v.at[j]], add=True)
                return 0
            lax.fori_loop(0, nb, eblk, 0)
            plsc.subcore_barrier()
            for t in range(5):
                pltpu.sync_copy(acc.at[pl.ds(sid * RT + t * 128, 128)],
                                rows_v)
                oi = cid if edge_split else ci
                pltpu.sync_copy(rows_v,
                                out.at[oi, pl.ds(sid * RT + t * 128, 128)])
            plsc.subcore_barrier()

    return prop


_prop_deg = _make_prop(16, 1, True, 40, ones_src=True)
_prop_es128 = _make_prop(128, 1, True, 40)
_prop_ch128 = _make_prop(128, 4, False, 79)


def _pad_edges_es(v, pad_val):
    per = 2 * 16 * 40 * 128
    vp = jnp.concatenate([v, jnp.full((per - E,), pad_val, jnp.int32)])
    return vp.reshape(2, 16, 40, 128)


def _pad_edges_ch(v, pad_val):
    per = 16 * 79 * 128
    vp = jnp.concatenate([v, jnp.full((per - E,), pad_val, jnp.int32)])
    return vp.reshape(16, 79, 128)


def _kpre_body(dp, x128, dis, y0):
    p = dp[0] + dp[1]
    deg = 1.0 + p[:, 0:1]
    d = lax.rsqrt(deg)
    dis[...] = d
    y0[...] = x128[...] * d


def _kpre(dp, x128):
    return pl.pallas_call(
        _kpre_body,
        grid=(N // R,),
        in_specs=[
            pl.BlockSpec((2, R, 16), lambda i: (0, i, 0)),
            pl.BlockSpec((R, 128), lambda i: (i, 0)),
        ],
        out_specs=[
            pl.BlockSpec((R, 1), lambda i: (i, 0)),
            pl.BlockSpec((R, 128), lambda i: (i, 0)),
        ],
        out_shape=[
            jax.ShapeDtypeStruct((N, 1), F32),
            jax.ShapeDtypeStruct((N, 128), F32),
        ],
    )(dp, x128)


def _k1_body(zp, y0, dis, w, b, out):
    d = dis[...]
    u = d * (zp[0] + zp[1] + y0[...])
    h = jnp.maximum(jnp.dot(u, w[...], preferred_element_type=F32) + b[...], 0.0)
    out[0] = d * h


def _k1(zp, y0, dis, w1p, b1):
    return pl.pallas_call(
        _k1_body,
        grid=(N // R, 4),
        in_specs=[
            pl.BlockSpec((2, R, 128), lambda i, c: (0, i, 0)),
            pl.BlockSpec((R, 128), lambda i, c: (i, 0)),
            pl.BlockSpec((R, 1), lambda i, c: (i, 0)),
            pl.BlockSpec((128, 128), lambda i, c: (0, c)),
            pl.BlockSpec((128,), lambda i, c: (c,)),
        ],
        out_specs=pl.BlockSpec((1, R, 128), lambda i, c: (c, i, 0)),
        out_shape=jax.ShapeDtypeStruct((4, N, 128), F32),
    )(zp, y0, dis, w1p, b1)


def _k2a_body(z1, y1, dis, w2r, b2, out):
    d = dis[...]
    acc = jnp.zeros((R, 1024), F32)
    for c in range(4):
        u = d * (z1[c] + y1[c])
        acc = acc + jnp.dot(u, w2r[c], preferred_element_type=F32)
    out[...] = jnp.maximum(acc + b2[...], 0.0)


def _k2a(z1, y1, dis, w2r, b2):
    return pl.pallas_call(
        _k2a_body,
        grid=(N // R,),
        in_specs=[
            pl.BlockSpec((4, R, 128), lambda i: (0, i, 0)),
            pl.BlockSpec((4, R, 128), lambda i: (0, i, 0)),
            pl.BlockSpec((R, 1), lambda i: (i, 0)),
            pl.BlockSpec((4, 128, 1024), lambda i: (0, 0, 0)),
            pl.BlockSpec((1024,), lambda i: (0,)),
        ],
        out_specs=pl.BlockSpec((R, 1024), lambda i: (i, 0)),
        out_shape=jax.ShapeDtypeStruct((N, 1024), F32),
    )(z1, y1, dis, w2r, b2)


def _k2b_body(h2, w3, dis, out):
    m = jnp.dot(h2[...], w3[...], preferred_element_type=F32)
    out[0] = dis[...] * m


def _k2b(h2, w3, dis):
    return pl.pallas_call(
        _k2b_body,
        grid=(N // R, 4),
        in_specs=[
            pl.BlockSpec((R, 1024), lambda i, c: (i, 0)),
            pl.BlockSpec((1024, 128), lambda i, c: (0, c)),
            pl.BlockSpec((R, 1), lambda i, c: (i, 0)),
        ],
        out_specs=pl.BlockSpec((1, R, 128), lambda i, c: (c, i, 0)),
        out_shape=jax.ShapeDtypeStruct((4, N, 128), F32),
    )(h2, w3, dis)


def _k3_body(z2, y2, dis, batch, b3, fc1w, fc1b, fc2p, fc2b, out,
             sums, cnt):
    i = pl.program_id(0)

    @pl.when(i == 0)
    def _init():
        sums[...] = jnp.zeros((G, 512), F32)
        cnt[...] = jnp.zeros((G, 8), F32)

    d = dis[...]
    oh = (batch[...] == lax.broadcasted_iota(jnp.int32, (R, G), 1)).astype(F32)
    dn = (((0,), (0,)), ((), ()))
    for c in range(4):
        u = d * (z2[c] + y2[c]) + b3[pl.ds(c * 128, 128)]
        h3c = jnp.maximum(u, 0.0)
        sums[:, pl.ds(c * 128, 128)] += lax.dot_general(
            oh, h3c, dn, preferred_element_type=F32)
    cnt[...] += lax.dot_general(oh, jnp.ones((R, 8), F32), dn,
                                preferred_element_type=F32)

    @pl.when(i == N // R - 1)
    def _fin():
        denom = jnp.maximum(cnt[:, 0:1], 1.0)
        g = sums[...] / denom
        a = jnp.maximum(
            jnp.dot(g, fc1w[...], preferred_element_type=F32) + fc1b[...], 0.0)
        out[...] = jnp.dot(a, fc2p[...], preferred_element_type=F32) + fc2b[...]


def _k3(z2, y2, dis, batchc, b3, fc1w, fc1b, fc2p, fc2b):
    return pl.pallas_call(
        _k3_body,
        grid=(N // R,),
        in_specs=[
            pl.BlockSpec((4, R, 128), lambda i: (0, i, 0)),
            pl.BlockSpec((4, R, 128), lambda i: (0, i, 0)),
            pl.BlockSpec((R, 1), lambda i: (i, 0)),
            pl.BlockSpec((R, 1), lambda i: (i, 0)),
            pl.BlockSpec((512,), lambda i: (0,)),
            pl.BlockSpec((512, 128), lambda i: (0, 0)),
            pl.BlockSpec((128,), lambda i: (0,)),
            pl.BlockSpec((128, 128), lambda i: (0, 0)),
            pl.BlockSpec((1,), lambda i: (0,)),
        ],
        out_specs=pl.BlockSpec((G, 128), lambda i: (0, 0)),
        out_shape=jax.ShapeDtypeStruct((G, 128), F32),
        scratch_shapes=[
            pltpu.VMEM((G, 512), F32),
            pltpu.VMEM((G, 8), F32),
        ],
    )(z2, y2, dis, batchc, b3, fc1w, fc1b, fc2p, fc2b)


def kernel(x, edge_index, batch, W1, b1, W2, b2, W3, b3,
           fc1_w, fc1_b, fc2_w, fc2_b):
    src = edge_index[0].astype(jnp.int32)
    dst = edge_index[1].astype(jnp.int32)
    src_es = _pad_edges_es(src, 0)
    dst_es = _pad_edges_es(dst, N)
    src_ch = _pad_edges_ch(src, 0)
    dst_ch = _pad_edges_ch(dst, N)

    dummy_y = jnp.zeros((8, 16), F32)
    x128 = jnp.pad(x, ((0, 0), (0, 125)))
    w1p = jnp.pad(W1, ((0, 125), (0, 0)))
    w2r = W2.reshape(4, 128, 1024)
    fc2p = jnp.pad(fc2_w, ((0, 0), (0, 127)))
    batchc = batch.astype(jnp.int32)[:, None]

    dp = _prop_deg(dummy_y, src_es, dst_es)
    dis, y0 = _kpre(dp, x128)
    zp = _prop_es128(y0, src_es, dst_es)
    y1 = _k1(zp, y0, dis, w1p, b1)
    z1 = _prop_ch128(y1.reshape(4 * N, 128), src_ch, dst_ch)
    h2 = _k2a(z1, y1, dis, w2r, b2)
    y2 = _k2b(h2, W3, dis)
    z2 = _prop_ch128(y2.reshape(4 * N, 128), src_ch, dst_ch)
    o = _k3(z2, y2, dis, batchc, b3, fc1_w, fc1_b, fc2p, fc2_b)
    return o[:, 0:1]


# speedup vs baseline: 9.5161x; 1.1761x over previous
import functools

import jax
import jax.numpy as jnp
from jax import lax
from jax.experimental import pallas as pl
from jax.experimental.pallas import tpu as pltpu
from jax.experimental.pallas import tpu_sc as plsc

N = 10000
E = 160000
G = 16
R = 1000
NP = 10240
RT = NP // 16
F32 = jnp.float32


B = 64


def _make_prop(F, n_passes, edge_split, nb, ones_src=False, bsz=None):
    bsz = B if bsz is None else bsz
    nhop = RT // bsz
    mesh = plsc.VectorSubcoreMesh(core_axis_name="c", subcore_axis_name="s")
    passes = 1 if edge_split else n_passes
    out_shape = (2 * passes, NP, F)
    scratch = [
        pltpu.VMEM((nb * bsz,), jnp.int32),
        pltpu.VMEM((nb, bsz), jnp.int32),
        pltpu.VMEM((bsz, F), F32),
        pltpu.VMEM((bsz, F), F32),
        pltpu.VMEM_SHARED((NP, F), F32),
        pltpu.SemaphoreType.DMA,
        pltpu.SemaphoreType.DMA,
    ]

    @functools.partial(
        pl.kernel, mesh=mesh,
        out_type=jax.ShapeDtypeStruct(out_shape, F32),
        scratch_types=scratch)
    def prop(y, srcb, dstb, zrows, out, src_v, dst_v, rows_v, rows_b, acc,
             gsa, gsb):
        cid = lax.axis_index("c")
        sid = lax.axis_index("s")
        if edge_split:
            if not ones_src:
                pltpu.sync_copy(srcb.at[cid, sid], src_v)
            pltpu.sync_copy(dstb.at[cid, sid], dst_v)
        else:
            pltpu.sync_copy(dstb.at[sid], dst_v)
        if ones_src:
            pltpu.sync_copy(y, rows_v)
        for p in range(passes):
            ci = cid * passes + p
            if not edge_split:
                pltpu.sync_copy(srcb.at[ci, sid], src_v)
            pltpu.sync_copy(zrows, acc.at[pl.ds(sid * RT, RT)])
            plsc.subcore_barrier()

            if ones_src:
                def eblk(j, _):
                    pltpu.sync_copy(rows_v, acc.at[dst_v.at[j]], add=True)
                    return 0
                lax.fori_loop(0, nb, eblk, 0)
            else:
                pltpu.async_copy(y.at[src_v.at[pl.ds(0, bsz)]], rows_v, gsa)

                def pair(jj, _):
                    j = 2 * jj
                    pltpu.make_async_copy(
                        y.at[src_v.at[pl.ds(j * bsz, bsz)]], rows_v,
                        gsa).wait()
                    pltpu.async_copy(
                        y.at[src_v.at[pl.ds((j + 1) * bsz, bsz)]], rows_b,
                        gsb)
                    pltpu.sync_copy(rows_v, acc.at[dst_v.at[j]], add=True)
                    pltpu.make_async_copy(
                        y.at[src_v.at[pl.ds((j + 1) * bsz, bsz)]], rows_b,
                        gsb).wait()

                    @pl.when(j + 2 < nb)
                    def _nx():
                        pltpu.async_copy(
                            y.at[src_v.at[pl.ds((j + 2) * bsz, bsz)]],
                            rows_v, gsa)
                    pltpu.sync_copy(rows_b, acc.at[dst_v.at[j + 1]],
                                    add=True)
                    return 0
                lax.fori_loop(0, nb // 2, pair, 0)
                if nb % 2:
                    pltpu.make_async_copy(
                        y.at[src_v.at[pl.ds((nb - 1) * bsz, bsz)]], rows_v,
                        gsa).wait()
                    pltpu.sync_copy(rows_v, acc.at[dst_v.at[nb - 1]],
                                    add=True)
            plsc.subcore_barrier()
            oi = cid if edge_split else ci
            for t in range(nhop):
                pltpu.sync_copy(acc.at[pl.ds(sid * RT + t * bsz, bsz)],
                                rows_v)
                pltpu.sync_copy(rows_v,
                                out.at[oi, pl.ds(sid * RT + t * bsz, bsz)])
            plsc.subcore_barrier()

    return prop


NB_ES = 79
NB_CH = 157

_prop_deg = _make_prop(128, 1, True, NB_ES, ones_src=True)
_prop_es128 = _make_prop(128, 1, True, NB_ES)
_prop_ch128 = _make_prop(128, 2, False, NB_CH)


def _pad_edges_es(v, pad_val, flat):
    per = 2 * 16 * NB_ES * B
    vp = jnp.concatenate([v, jnp.full((per - E,), pad_val, jnp.int32)])
    shape = (2, 16, NB_ES * B) if flat else (2, 16, NB_ES, B)
    return vp.reshape(shape)


def _pad_edges_ch(v, pad_val, flat, n_chunks=1):
    per = 16 * NB_CH * B
    vp = jnp.concatenate([v, jnp.full((per - E,), pad_val, jnp.int32)])
    if not flat:
        return vp.reshape(16, NB_CH, B)
    vpc = vp.reshape(1, 16, NB_CH * B)
    offs = (jnp.arange(n_chunks, dtype=jnp.int32) * N).reshape(-1, 1, 1)
    return vpc + offs


def _kpre_body(dp, x128, dis, y0):
    p = dp[0] + dp[1]
    deg = 1.0 + p[:, 0:1]
    d = lax.rsqrt(deg)
    dis[...] = d
    y0[...] = x128[...] * d


def _kpre(dp, x128):
    return pl.pallas_call(
        _kpre_body,
        grid=(N // R,),
        in_specs=[
            pl.BlockSpec((2, R, 128), lambda i: (0, i, 0)),
            pl.BlockSpec((R, 128), lambda i: (i, 0)),
        ],
        out_specs=[
            pl.BlockSpec((R, 1), lambda i: (i, 0)),
            pl.BlockSpec((R, 128), lambda i: (i, 0)),
        ],
        out_shape=[
            jax.ShapeDtypeStruct((N, 1), F32),
            jax.ShapeDtypeStruct((N, 128), F32),
        ],
    )(dp, x128)


def _k1_body(zp, y0, dis, w, b, out):
    d = dis[...]
    u = d * (zp[0] + zp[1] + y0[...])
    h = jnp.maximum(jnp.dot(u, w[...], preferred_element_type=F32) + b[...], 0.0)
    out[0] = d * h


def _k1(zp, y0, dis, w1p, b1):
    return pl.pallas_call(
        _k1_body,
        grid=(N // R, 4),
        in_specs=[
            pl.BlockSpec((2, R, 128), lambda i, c: (0, i, 0)),
            pl.BlockSpec((R, 128), lambda i, c: (i, 0)),
            pl.BlockSpec((R, 1), lambda i, c: (i, 0)),
            pl.BlockSpec((128, 128), lambda i, c: (0, c)),
            pl.BlockSpec((128,), lambda i, c: (c,)),
        ],
        out_specs=pl.BlockSpec((1, R, 128), lambda i, c: (c, i, 0)),
        out_shape=jax.ShapeDtypeStruct((4, N, 128), F32),
    )(zp, y0, dis, w1p, b1)


def _k2a_body(z1, y1, dis, w2r, b2, out):
    d = dis[...]
    acc = jnp.zeros((R, 1024), F32)
    for c in range(4):
        u = d * (z1[c] + y1[c])
        acc = acc + jnp.dot(u, w2r[c], preferred_element_type=F32)
    out[...] = jnp.maximum(acc + b2[...], 0.0)


def _k2a(z1, y1, dis, w2r, b2):
    return pl.pallas_call(
        _k2a_body,
        grid=(N // R,),
        in_specs=[
            pl.BlockSpec((4, R, 128), lambda i: (0, i, 0)),
            pl.BlockSpec((4, R, 128), lambda i: (0, i, 0)),
            pl.BlockSpec((R, 1), lambda i: (i, 0)),
            pl.BlockSpec((4, 128, 1024), lambda i: (0, 0, 0)),
            pl.BlockSpec((1024,), lambda i: (0,)),
        ],
        out_specs=pl.BlockSpec((R, 1024), lambda i: (i, 0)),
        out_shape=jax.ShapeDtypeStruct((N, 1024), F32),
    )(z1, y1, dis, w2r, b2)


def _k2b_body(h2, w3, dis, out):
    m = jnp.dot(h2[...], w3[...], preferred_element_type=F32)
    out[0] = dis[...] * m


def _k2b(h2, w3, dis):
    return pl.pallas_call(
        _k2b_body,
        grid=(N // R, 4),
        in_specs=[
            pl.BlockSpec((R, 1024), lambda i, c: (i, 0)),
            pl.BlockSpec((1024, 128), lambda i, c: (0, c)),
            pl.BlockSpec((R, 1), lambda i, c: (i, 0)),
        ],
        out_specs=pl.BlockSpec((1, R, 128), lambda i, c: (c, i, 0)),
        out_shape=jax.ShapeDtypeStruct((4, N, 128), F32),
    )(h2, w3, dis)


def _k3_body(z2, y2, dis, batch, b3, fc1w, fc1b, fc2p, fc2b, out,
             sums, cnt):
    i = pl.program_id(0)

    @pl.when(i == 0)
    def _init():
        sums[...] = jnp.zeros((G, 512), F32)
        cnt[...] = jnp.zeros((G, 8), F32)

    d = dis[...]
    oh = (batch[...] == lax.broadcasted_iota(jnp.int32, (R, G), 1)).astype(F32)
    dn = (((0,), (0,)), ((), ()))
    for c in range(4):
        u = d * (z2[c] + y2[c]) + b3[pl.ds(c * 128, 128)]
        h3c = jnp.maximum(u, 0.0)
        sums[:, pl.ds(c * 128, 128)] += lax.dot_general(
            oh, h3c, dn, preferred_element_type=F32)
    cnt[...] += lax.dot_general(oh, jnp.ones((R, 8), F32), dn,
                                preferred_element_type=F32)

    @pl.when(i == N // R - 1)
    def _fin():
        denom = jnp.maximum(cnt[:, 0:1], 1.0)
        g = sums[...] / denom
        a = jnp.maximum(
            jnp.dot(g, fc1w[...], preferred_element_type=F32) + fc1b[...], 0.0)
        out[...] = jnp.dot(a, fc2p[...], preferred_element_type=F32) + fc2b[...]


def _k3(z2, y2, dis, batchc, b3, fc1w, fc1b, fc2p, fc2b):
    return pl.pallas_call(
        _k3_body,
        grid=(N // R,),
        in_specs=[
            pl.BlockSpec((4, R, 128), lambda i: (0, i, 0)),
            pl.BlockSpec((4, R, 128), lambda i: (0, i, 0)),
            pl.BlockSpec((R, 1), lambda i: (i, 0)),
            pl.BlockSpec((R, 1), lambda i: (i, 0)),
            pl.BlockSpec((512,), lambda i: (0,)),
            pl.BlockSpec((512, 128), lambda i: (0, 0)),
            pl.BlockSpec((128,), lambda i: (0,)),
            pl.BlockSpec((128, 128), lambda i: (0, 0)),
            pl.BlockSpec((1,), lambda i: (0,)),
        ],
        out_specs=pl.BlockSpec((G, 128), lambda i: (0, 0)),
        out_shape=jax.ShapeDtypeStruct((G, 128), F32),
        scratch_shapes=[
            pltpu.VMEM((G, 512), F32),
            pltpu.VMEM((G, 8), F32),
        ],
    )(z2, y2, dis, batchc, b3, fc1w, fc1b, fc2p, fc2b)


def kernel(x, edge_index, batch, W1, b1, W2, b2, W3, b3,
           fc1_w, fc1_b, fc2_w, fc2_b):
    src = edge_index[0].astype(jnp.int32)
    dst = edge_index[1].astype(jnp.int32)
    src_es = _pad_edges_es(src, 0, True)
    dst_es = _pad_edges_es(dst, N, False)
    src_ch = _pad_edges_ch(src, 0, True, 4)
    dst_ch = _pad_edges_ch(dst, N, False)
    z128 = jnp.zeros((RT, 128), F32)
    ones_blk = jnp.ones((B, 128), F32)

    x128 = jnp.pad(x, ((0, 0), (0, 125)))
    w1p = jnp.pad(W1, ((0, 125), (0, 0)))
    w2r = W2.reshape(4, 128, 1024)
    fc2p = jnp.pad(fc2_w, ((0, 0), (0, 127)))
    batchc = batch.astype(jnp.int32)[:, None]

    dp = _prop_deg(ones_blk, src_es, dst_es, z128)
    dis, y0 = _kpre(dp, x128)
    zp = _prop_es128(y0, src_es, dst_es, z128)
    y1 = _k1(zp, y0, dis, w1p, b1)
    z1 = _prop_ch128(y1.reshape(4 * N, 128), src_ch, dst_ch, z128)
    h2 = _k2a(z1, y1, dis, w2r, b2)
    y2 = _k2b(h2, W3, dis)
    z2 = _prop_ch128(y2.reshape(4 * N, 128), src_ch, dst_ch, z128)
    o = _k3(z2, y2, dis, batchc, b3, fc1_w, fc1_b, fc2p, fc2_b)
    return o[:, 0:1]
